# baseline (speedup 1.0000x reference)
import functools

import jax
import jax.numpy as jnp
from jax import lax
from jax.experimental import pallas as pl
from jax.experimental.pallas import tpu as pltpu
from jax.experimental.pallas import tpu_sc as plsc

_D = 32
_NB = 16384
_NT = 50
_NC = 2
_NS = 16
_NW = _NC * _NS
_BW = _NB // _NW


@functools.partial(
    pl.kernel,
    mesh=plsc.VectorSubcoreMesh(core_axis_name="c", subcore_axis_name="s"),
    out_type=jax.ShapeDtypeStruct((_NT, _D, _NB, 1), jnp.float32),
    scratch_types=[
        pltpu.VMEM((2, _BW), jnp.int32),
        pltpu.VMEM((2, _BW, _D), jnp.float32),
        pltpu.SemaphoreType.DMA,
        pltpu.SemaphoreType.DMA,
        pltpu.SemaphoreType.DMA,
        pltpu.SemaphoreType.DMA,
        pltpu.SemaphoreType.DMA,
    ],
    compiler_params=pltpu.CompilerParams(use_tc_tiling_on_sc=False),
)
def _gather_kernel(ids_hbm, table_hbm, out_hbm, idx_v, rows_v, si0, si1, sg, ss0, ss1):
    wid = lax.axis_index("s") * _NC + lax.axis_index("c")
    b0 = wid * _BW
    si = (si0, si1)
    ss = (ss0, ss1)

    pltpu.async_copy(ids_hbm.at[0, pl.ds(b0, _BW)], idx_v.at[0], si0)

    def outer(tt, carry):
        for k in range(2):
            t = tt * 2 + k

            @pl.when(t < _NT - 1)
            def _prefetch():
                pltpu.async_copy(
                    ids_hbm.at[t + 1, pl.ds(b0, _BW)], idx_v.at[1 - k], si[1 - k]
                )

            pltpu.make_async_copy(
                ids_hbm.at[t, pl.ds(b0, _BW)], idx_v.at[k], si[k]
            ).wait()

            @pl.when(t >= 2)
            def _drain():
                for d in range(_D):
                    pltpu.make_async_copy(
                        rows_v.at[k, pl.ds(0, _BW), pl.ds(d, 1)],
                        out_hbm.at[t, d, pl.ds(b0, _BW), pl.ds(0, 1)],
                        ss[k],
                    ).wait()

            pltpu.async_copy(table_hbm.at[idx_v.at[k]], rows_v.at[k], sg).wait()

            for d in range(_D):
                pltpu.async_copy(
                    rows_v.at[k, pl.ds(0, _BW), pl.ds(d, 1)],
                    out_hbm.at[t, d, pl.ds(b0, _BW), pl.ds(0, 1)],
                    ss[k],
                )
        return carry

    lax.fori_loop(0, _NT // 2, outer, 0)

    for k, t in ((0, _NT - 2), (1, _NT - 1)):
        for d in range(_D):
            pltpu.make_async_copy(
                rows_v.at[k, pl.ds(0, _BW), pl.ds(d, 1)],
                out_hbm.at[t, d, pl.ds(b0, _BW), pl.ds(0, 1)],
                ss[k],
            ).wait()


def kernel(token_ids, embedding):
    ids_t = jnp.transpose(token_ids).astype(jnp.int32)
    out_t = _gather_kernel(ids_t, embedding)
    out_t = jnp.squeeze(out_t, -1)
    return jnp.transpose(out_t, (2, 0, 1))


# speedup vs baseline: 62.5772x; 62.5772x over previous
import functools

import jax
import jax.numpy as jnp
from jax import lax
from jax.experimental import pallas as pl
from jax.experimental.pallas import tpu as pltpu
from jax.experimental.pallas import tpu_sc as plsc

_D = 32
_NB = 16384
_NT = 50
_NC = 2
_NS = 16
_NW = _NC * _NS
_BW = _NB // _NW


@functools.partial(
    pl.kernel,
    mesh=plsc.VectorSubcoreMesh(core_axis_name="c", subcore_axis_name="s"),
    out_type=jax.ShapeDtypeStruct((_NT, _NB, _D), jnp.float32),
    scratch_types=[
        pltpu.VMEM((2, _BW), jnp.int32),
        pltpu.VMEM((2, _BW, _D), jnp.float32),
        pltpu.SemaphoreType.DMA,
        pltpu.SemaphoreType.DMA,
        pltpu.SemaphoreType.DMA,
        pltpu.SemaphoreType.DMA,
        pltpu.SemaphoreType.DMA,
    ],
    compiler_params=pltpu.CompilerParams(use_tc_tiling_on_sc=False),
)
def _gather_kernel(ids_hbm, table_hbm, out_hbm, idx_v, rows_v, si0, si1, sg, ss0, ss1):
    wid = lax.axis_index("s") * _NC + lax.axis_index("c")
    b0 = wid * _BW
    si = (si0, si1)
    ss = (ss0, ss1)

    pltpu.async_copy(ids_hbm.at[0, pl.ds(b0, _BW)], idx_v.at[0], si0)

    def outer(tt, carry):
        for k in range(2):
            t = tt * 2 + k

            @pl.when(t < _NT - 1)
            def _prefetch():
                pltpu.async_copy(
                    ids_hbm.at[t + 1, pl.ds(b0, _BW)], idx_v.at[1 - k], si[1 - k]
                )

            pltpu.make_async_copy(
                ids_hbm.at[t, pl.ds(b0, _BW)], idx_v.at[k], si[k]
            ).wait()

            @pl.when(t >= 2)
            def _drain():
                pltpu.make_async_copy(
                    rows_v.at[k], out_hbm.at[t, pl.ds(b0, _BW)], ss[k]
                ).wait()

            pltpu.async_copy(table_hbm.at[idx_v.at[k]], rows_v.at[k], sg).wait()

            pltpu.async_copy(rows_v.at[k], out_hbm.at[t, pl.ds(b0, _BW)], ss[k])
        return carry

    lax.fori_loop(0, _NT // 2, outer, 0)

    pltpu.make_async_copy(rows_v.at[0], out_hbm.at[_NT - 2, pl.ds(b0, _BW)], ss0).wait()
    pltpu.make_async_copy(rows_v.at[1], out_hbm.at[_NT - 1, pl.ds(b0, _BW)], ss1).wait()


def kernel(token_ids, embedding):
    ids_t = jnp.transpose(token_ids).astype(jnp.int32)
    out_t = _gather_kernel(ids_t, embedding)
    return jnp.transpose(out_t, (1, 0, 2))
